# baseline (speedup 1.0000x reference)
import functools

import jax
import jax.numpy as jnp
from jax import lax
from jax.experimental import pallas as pl
from jax.experimental.pallas import tpu as pltpu

B, C, N, T = 2, 16, 1000, 10
E, H = 16, 8
D = E * T


def _conv_kernel(x_ref, ws_ref, wc_ref, bs_ref, bc_ref, out_ref):
    x = x_ref[...]
    x1 = jnp.dot(x, ws_ref[...], preferred_element_type=jnp.float32) + bs_ref[...]
    x2 = jnp.dot(x, wc_ref[...], preferred_element_type=jnp.float32) + bc_ref[...]
    x2 = jnp.where(x2 >= 0, x2, 0.01 * x2)
    out_ref[...] = x1 + x2


def _gat_kernel(with_res, *refs):
    if with_res:
        h_ref, w_ref, al_ref, ar_ref, bias_ref, gt_ref, res_ref, out_ref = refs
    else:
        h_ref, w_ref, al_ref, ar_ref, bias_ref, gt_ref, out_ref = refs
        res_ref = None
    hh = pl.program_id(1)

    h_b = h_ref[0]
    w = w_ref[0]
    feat = jnp.dot(h_b, w, preferred_element_type=jnp.float32)

    al = al_ref[0]
    ar = ar_ref[0]
    dnum = (((1,), (1,)), ((), ()))
    el_row = lax.dot_general(al, feat, dnum, preferred_element_type=jnp.float32)
    er_col = lax.dot_general(feat, ar, dnum, preferred_element_type=jnp.float32)

    s = er_col + el_row
    s = jnp.where(s >= 0, s, 0.2 * s)
    mask = gt_ref[...] != 0.0
    s = jnp.where(mask, s, -1e30)
    m = jnp.max(s, axis=1, keepdims=True)
    p = jnp.exp(s - m)
    z = jnp.sum(p, axis=1, keepdims=True)
    alpha = p / z
    alpha = jnp.where(mask, alpha, 0.0)

    rst = jnp.dot(alpha.astype(jnp.bfloat16), feat.astype(jnp.bfloat16),
                  preferred_element_type=jnp.float32)
    rst = rst + bias_ref[0]
    rst = jnp.where(rst > 0, rst, jnp.exp(rst) - 1.0)
    acc = rst * (1.0 / H)

    @pl.when(hh == 0)
    def _init():
        if res_ref is None:
            out_ref[0] = acc
        else:
            out_ref[0] = res_ref[0] + acc

    @pl.when(hh != 0)
    def _acc():
        out_ref[0] = out_ref[0] + acc


def _gat_layer(h, W, al, ar, bias, gt, res=None):
    W_r = W.reshape(D, H, D).transpose(1, 0, 2)
    bias_r = bias.reshape(H, 1, D)
    al_r = al.reshape(H, 1, D)
    ar_r = ar.reshape(H, 1, D)
    inputs = [h, W_r, al_r, ar_r, bias_r, gt]
    in_specs = [
        pl.BlockSpec((1, N, D), lambda b, hh: (b, 0, 0)),
        pl.BlockSpec((1, D, D), lambda b, hh: (hh, 0, 0)),
        pl.BlockSpec((1, 1, D), lambda b, hh: (hh, 0, 0)),
        pl.BlockSpec((1, 1, D), lambda b, hh: (hh, 0, 0)),
        pl.BlockSpec((1, 1, D), lambda b, hh: (hh, 0, 0)),
        pl.BlockSpec((N, N), lambda b, hh: (0, 0)),
    ]
    if res is not None:
        inputs.append(res)
        in_specs.append(pl.BlockSpec((1, N, D), lambda b, hh: (b, 0, 0)))
    return pl.pallas_call(
        functools.partial(_gat_kernel, res is not None),
        grid=(B, H),
        in_specs=in_specs,
        out_specs=pl.BlockSpec((1, N, D), lambda b, hh: (b, 0, 0)),
        out_shape=jax.ShapeDtypeStruct((B, N, D), jnp.float32),
        compiler_params=pltpu.CompilerParams(
            dimension_semantics=("parallel", "arbitrary")),
    )(*inputs)


def kernel(x, g, w_start, b_start, w_cat, b_cat, W1, al1, ar1, bias1, W2, al2, ar2, bias2):
    X = x.transpose(0, 2, 1, 3).reshape(B * N, C * T)
    eye_t = jnp.eye(T, dtype=jnp.float32)
    Wbs = jnp.einsum('ec,tu->cteu', w_start, eye_t).reshape(C * T, E * T)
    Wbc = jnp.einsum('ec,tu->cteu', w_cat, eye_t).reshape(C * T, E * T)
    bs = jnp.repeat(b_start, T).reshape(1, E * T)
    bc = jnp.repeat(b_cat, T).reshape(1, E * T)
    gt = g.T

    xs_flat = pl.pallas_call(
        _conv_kernel,
        out_shape=jax.ShapeDtypeStruct((B * N, E * T), jnp.float32),
    )(X, Wbs, Wbc, bs, bc)

    h0 = xs_flat.reshape(B, N, D)
    h1 = _gat_layer(h0, W1, al1, ar1, bias1, gt)
    h2 = _gat_layer(h1, W2, al2, ar2, bias2, gt, res=h0)

    out = h2.reshape(B, N, E, T).transpose(0, 2, 1, 3)
    return out


# speedup vs baseline: 1.1890x; 1.1890x over previous
import functools

import jax
import jax.numpy as jnp
from jax import lax
from jax.experimental import pallas as pl
from jax.experimental.pallas import tpu as pltpu

B, C, N, T = 2, 16, 1000, 10
E, H = 16, 8
D = E * T


def _conv_kernel(x_ref, ws_ref, wc_ref, bs_ref, bc_ref, out_ref):
    x = x_ref[...]
    x1 = jnp.dot(x, ws_ref[...], preferred_element_type=jnp.float32) + bs_ref[...]
    x2 = jnp.dot(x, wc_ref[...], preferred_element_type=jnp.float32) + bc_ref[...]
    x2 = jnp.where(x2 >= 0, x2, 0.01 * x2)
    out_ref[...] = x1 + x2


def _gat_kernel(with_res, *refs):
    if with_res:
        h_ref, w_ref, al_ref, ar_ref, bias_ref, gt_ref, res_ref, out_ref, mask_ref = refs
    else:
        h_ref, w_ref, al_ref, ar_ref, bias_ref, gt_ref, out_ref, mask_ref = refs
        res_ref = None
    hh = pl.program_id(1)
    bf = jnp.bfloat16

    @pl.when(hh == 0)
    def _mk_mask():
        mask_ref[...] = jnp.where(gt_ref[...] != 0.0, 1.0, 0.0).astype(bf)

    h_b = h_ref[0].astype(bf)
    w = w_ref[0].astype(bf)
    feat32 = jnp.dot(h_b, w, preferred_element_type=jnp.float32)
    feat = feat32.astype(bf)

    al = al_ref[0]
    ar = ar_ref[0]
    dnum = (((1,), (1,)), ((), ()))
    el_row = lax.dot_general(al, feat32, dnum, preferred_element_type=jnp.float32)
    er_col = lax.dot_general(feat32, ar, dnum, preferred_element_type=jnp.float32)

    m_el = jnp.max(el_row, axis=1, keepdims=True)
    e1_row = jnp.exp(el_row - m_el).astype(bf)
    e2_row = jnp.exp(0.2 * (el_row - m_el)).astype(bf)
    u = er_col + m_el
    cstab = jnp.maximum(u, 0.2 * u)
    f1_col = jnp.exp(u - cstab).astype(bf)
    f2_col = jnp.exp(0.2 * u - cstab).astype(bf)

    er_col_bf = er_col.astype(bf)
    nel_row_bf = (-el_row).astype(bf)
    cond = er_col_bf >= nel_row_bf
    num = jnp.where(cond, f1_col * e1_row, f2_col * e2_row) * mask_ref[...]

    rstq = jnp.dot(num, feat, preferred_element_type=jnp.float32)
    z = jnp.dot(num, jnp.ones((N, 1), bf), preferred_element_type=jnp.float32)
    z = jnp.maximum(z, 1e-30)

    rst = rstq * (1.0 / z) + bias_ref[0]
    rst = jnp.where(rst > 0, rst, jnp.exp(rst) - 1.0)
    acc = rst * (1.0 / H)

    @pl.when(hh == 0)
    def _init():
        if res_ref is None:
            out_ref[0] = acc
        else:
            out_ref[0] = res_ref[0] + acc

    @pl.when(hh != 0)
    def _acc():
        out_ref[0] = out_ref[0] + acc


def _gat_layer(h, W, al, ar, bias, gt, res=None):
    W_r = W.reshape(D, H, D).transpose(1, 0, 2)
    bias_r = bias.reshape(H, 1, D)
    al_r = al.reshape(H, 1, D)
    ar_r = ar.reshape(H, 1, D)
    inputs = [h, W_r, al_r, ar_r, bias_r, gt]
    in_specs = [
        pl.BlockSpec((1, N, D), lambda b, hh: (b, 0, 0)),
        pl.BlockSpec((1, D, D), lambda b, hh: (hh, 0, 0)),
        pl.BlockSpec((1, 1, D), lambda b, hh: (hh, 0, 0)),
        pl.BlockSpec((1, 1, D), lambda b, hh: (hh, 0, 0)),
        pl.BlockSpec((1, 1, D), lambda b, hh: (hh, 0, 0)),
        pl.BlockSpec((N, N), lambda b, hh: (0, 0)),
    ]
    if res is not None:
        inputs.append(res)
        in_specs.append(pl.BlockSpec((1, N, D), lambda b, hh: (b, 0, 0)))
    return pl.pallas_call(
        functools.partial(_gat_kernel, res is not None),
        grid=(B, H),
        in_specs=in_specs,
        out_specs=pl.BlockSpec((1, N, D), lambda b, hh: (b, 0, 0)),
        out_shape=jax.ShapeDtypeStruct((B, N, D), jnp.float32),
        scratch_shapes=[pltpu.VMEM((N, N), jnp.bfloat16)],
        compiler_params=pltpu.CompilerParams(
            dimension_semantics=("parallel", "arbitrary")),
    )(*inputs)


def kernel(x, g, w_start, b_start, w_cat, b_cat, W1, al1, ar1, bias1, W2, al2, ar2, bias2):
    X = x.transpose(0, 2, 1, 3).reshape(B * N, C * T)
    eye_t = jnp.eye(T, dtype=jnp.float32)
    Wbs = jnp.einsum('ec,tu->cteu', w_start, eye_t).reshape(C * T, E * T)
    Wbc = jnp.einsum('ec,tu->cteu', w_cat, eye_t).reshape(C * T, E * T)
    bs = jnp.repeat(b_start, T).reshape(1, E * T)
    bc = jnp.repeat(b_cat, T).reshape(1, E * T)
    gt = g.T

    xs_flat = pl.pallas_call(
        _conv_kernel,
        out_shape=jax.ShapeDtypeStruct((B * N, E * T), jnp.float32),
    )(X, Wbs, Wbc, bs, bc)

    h0 = xs_flat.reshape(B, N, D)
    h1 = _gat_layer(h0, W1, al1, ar1, bias1, gt)
    h2 = _gat_layer(h1, W2, al2, ar2, bias2, gt, res=h0)

    out = h2.reshape(B, N, E, T).transpose(0, 2, 1, 3)
    return out


# speedup vs baseline: 2.0164x; 1.6958x over previous
import jax
import jax.numpy as jnp
from jax.experimental import pallas as pl
from jax.experimental.pallas import tpu as pltpu

B, C, N, T = 2, 16, 1000, 10
E, H = 16, 8
D = E * T
DE = D + 1
HP = 256


def _fused_kernel(x_ref, gt_ref, wbs_ref, wbc_ref, bconv_ref,
                  w1_ref, wal1_ref, war1_ref, b1_ref,
                  w2_ref, wal2_ref, war2_ref, b2_ref, out_ref):
    bf = jnp.bfloat16
    x = x_ref[0]
    x1 = jnp.dot(x, wbs_ref[...], preferred_element_type=jnp.float32) + bconv_ref[0:1, :]
    x2 = jnp.dot(x, wbc_ref[...], preferred_element_type=jnp.float32) + bconv_ref[1:2, :]
    x2 = jnp.where(x2 >= 0, x2, 0.01 * x2)
    h0 = x1 + x2
    mask = jnp.where(gt_ref[...] != 0.0, 1.0, 0.0).astype(bf)

    def layer(h32, w_ref_, wal_ref_, war_ref_, b_ref_):
        h_bf = h32.astype(bf)
        el_all = jnp.dot(h32, wal_ref_[...], preferred_element_type=jnp.float32)
        er_all = jnp.dot(h32, war_ref_[...], preferred_element_type=jnp.float32)
        elT = el_all.T
        m_col = jnp.max(elT, axis=1, keepdims=True)
        e1T = jnp.exp(elT - m_col).astype(bf)
        e2T = jnp.exp(0.2 * (elT - m_col)).astype(bf)
        nelT = (-elT).astype(bf)
        m_row = jnp.max(el_all, axis=0, keepdims=True)
        u = er_all + m_row
        cst = jnp.maximum(u, 0.2 * u)
        f1 = jnp.exp(u - cst).astype(bf)
        f2 = jnp.exp(0.2 * u - cst).astype(bf)
        er_bf = er_all.astype(bf)
        acc = None
        for hh in range(H):
            fb = jnp.dot(h_bf, w_ref_[:, HP * hh:HP * (hh + 1)],
                         preferred_element_type=jnp.float32).astype(bf)
            cond = er_bf[:, hh:hh + 1] >= nelT[hh:hh + 1, :]
            num = jnp.where(cond,
                            f1[:, hh:hh + 1] * e1T[hh:hh + 1, :],
                            f2[:, hh:hh + 1] * e2T[hh:hh + 1, :]) * mask
            rq = jnp.dot(num, fb, preferred_element_type=jnp.float32)
            z = jnp.maximum(rq[:, D:D + 1], 1e-30)
            rst = rq[:, :DE] * (1.0 / z) + b_ref_[hh:hh + 1, :]
            rst = jnp.where(rst > 0, rst, jnp.exp(rst) - 1.0)
            a = rst * (1.0 / H)
            acc = a if acc is None else acc + a
        return acc

    h1 = layer(h0, w1_ref, wal1_ref, war1_ref, b1_ref)
    h2 = layer(h1, w2_ref, wal2_ref, war2_ref, b2_ref)
    out_ref[0] = h0[:, :D] + h2[:, :D]


def kernel(x, g, w_start, b_start, w_cat, b_cat, W1, al1, ar1, bias1, W2, al2, ar2, bias2):
    f32 = jnp.float32
    X = x.transpose(0, 2, 1, 3).reshape(B, N, C * T)
    X_ext = jnp.concatenate([X, jnp.ones((B, N, 1), f32)], axis=2)
    gt = g.T

    eye_t = jnp.eye(T, dtype=f32)

    def conv_block(w):
        return jnp.einsum('ec,tu->cteu', w, eye_t).reshape(C * T, E * T)

    wbs = jnp.zeros((DE, DE), f32).at[:D, :D].set(conv_block(w_start)).at[D, D].set(1.0)
    wbc = jnp.zeros((DE, DE), f32).at[:D, :D].set(conv_block(w_cat))
    bconv = jnp.stack([
        jnp.concatenate([jnp.repeat(b_start, T), jnp.zeros((1,), f32)]),
        jnp.concatenate([jnp.repeat(b_cat, T), jnp.zeros((1,), f32)]),
    ])

    def head_blocks(W):
        Wr = W.reshape(D, H, D).transpose(1, 0, 2)
        Wp = jnp.zeros((H, DE, HP), f32)
        Wp = Wp.at[:, :D, :D].set(Wr).at[:, D, D].set(1.0)
        return Wp.transpose(1, 0, 2).reshape(DE, H * HP).astype(jnp.bfloat16)

    def fold_attn(W, a):
        wal = jnp.einsum('dhe,he->dh', W.reshape(D, H, D), a)
        return jnp.concatenate([wal, jnp.zeros((1, H), f32)], axis=0)

    def bias_ext(bias):
        return jnp.concatenate(
            [bias.reshape(H, D), jnp.zeros((H, 1), f32)], axis=1)

    w1b, w2b = head_blocks(W1), head_blocks(W2)
    wal1, war1 = fold_attn(W1, al1), fold_attn(W1, ar1)
    wal2, war2 = fold_attn(W2, al2), fold_attn(W2, ar2)
    b1e, b2e = bias_ext(bias1), bias_ext(bias2)

    const = lambda *shape: pl.BlockSpec(shape, lambda b: tuple(0 for _ in shape))
    out = pl.pallas_call(
        _fused_kernel,
        grid=(B,),
        in_specs=[
            pl.BlockSpec((1, N, DE), lambda b: (b, 0, 0)),
            const(N, N),
            const(DE, DE), const(DE, DE), const(2, DE),
            const(DE, H * HP), const(DE, H), const(DE, H), const(H, DE),
            const(DE, H * HP), const(DE, H), const(DE, H), const(H, DE),
        ],
        out_specs=pl.BlockSpec((1, N, D), lambda b: (b, 0, 0)),
        out_shape=jax.ShapeDtypeStruct((B, N, D), f32),
        compiler_params=pltpu.CompilerParams(dimension_semantics=("parallel",)),
    )(X_ext, gt, wbs, wbc, bconv, w1b, wal1, war1, b1e, w2b, wal2, war2, b2e)

    return out.reshape(B, N, E, T).transpose(0, 2, 1, 3)
